# Initial kernel scaffold; baseline (speedup 1.0000x reference)
import jax
import jax.numpy as jnp
from jax.experimental import pallas as pl


def kernel(cri_out, net_out, class_id):
    raise NotImplementedError("write your pallas kernel here")


# speedup vs baseline: 1.7767x; 1.7767x over previous
import jax
import jax.numpy as jnp
from jax.experimental import pallas as pl
from jax.experimental.pallas import tpu as pltpu

_B = 128
_V = 100000
_BN = 12800
_NB = (_V + _BN - 1) // _BN


def _acc_body(cid_ref, x_ref, out_ref, t_ref, cnt_ref):
    p = pl.program_id(0)
    j = pl.program_id(1)

    x = x_ref[...]
    cid = cid_ref[...]
    cols = jax.lax.broadcasted_iota(jnp.int32, (_B, _BN), 1) + j * _BN

    @pl.when(p == 0)
    def _gather():
        @pl.when(j == 0)
        def _init():
            t_ref[...] = jnp.zeros_like(t_ref)
            cnt_ref[...] = jnp.zeros_like(cnt_ref)

        hit = cols == cid
        t_ref[...] += jnp.sum(jnp.where(hit, x, 0.0), axis=1, keepdims=True)

    @pl.when(p == 1)
    def _count():
        t = t_ref[...]
        ahead = ((x > t) | ((x == t) & (cols < cid))) & (cols < _V)
        cnt_ref[...] += jnp.sum(
            jnp.where(ahead, 1.0, 0.0), axis=1, keepdims=True
        )

        @pl.when(j == _NB - 1)
        def _final():
            cnt = cnt_ref[...]
            top1 = jnp.sum(jnp.where(cnt < 1.0, 1.0, 0.0))
            top5 = jnp.sum(jnp.where(cnt < 5.0, 1.0, 0.0))
            out_ref[0] = top1 * (100.0 / _B)
            out_ref[1] = top5 * (100.0 / _B)


def kernel(cri_out, net_out, class_id):
    del cri_out
    cid = class_id.astype(jnp.int32).reshape(_B, 1)
    return pl.pallas_call(
        _acc_body,
        grid=(2, _NB),
        in_specs=[
            pl.BlockSpec((_B, 1), lambda p, j: (0, 0)),
            pl.BlockSpec((_B, _BN), lambda p, j: (0, j)),
        ],
        out_specs=pl.BlockSpec(memory_space=pltpu.SMEM),
        out_shape=jax.ShapeDtypeStruct((2,), jnp.float32),
        scratch_shapes=[
            pltpu.VMEM((_B, 1), jnp.float32),
            pltpu.VMEM((_B, 1), jnp.float32),
        ],
    )(cid, net_out)
